# Initial kernel scaffold; baseline (speedup 1.0000x reference)
import jax
import jax.numpy as jnp
from jax.experimental import pallas as pl


def kernel(x, transform):
    raise NotImplementedError("write your pallas kernel here")


# speedup vs baseline: 5.0758x; 5.0758x over previous
import functools

import jax
import jax.numpy as jnp
from jax import lax
from jax.experimental import pallas as pl
from jax.experimental.pallas import tpu as pltpu
from jax.experimental.pallas import tpu_sc as plsc

H = 224
W = 224
C = 96
B = 8
NPIX = H * W
NPLANES = B * C
NLANES = 16
NWORKERS = 32
WORKERS_PER_IMAGE = NWORKERS // B
CH_PER_WORKER = C // WORKERS_PER_IMAGE
CHUNKS_PER_ROW = W // NLANES


def _splat(vec, lane):
    return jnp.full((NLANES,), vec[lane], dtype=jnp.float32)


def _bf16_round(v):
    u = plsc.bitcast(v, jnp.uint32)
    r = (u + jnp.uint32(0x7FFF) + ((u >> jnp.uint32(16)) & jnp.uint32(1))) \
        & jnp.uint32(0xFFFF0000)
    return plsc.bitcast(r, jnp.float32)


def _affine_body(x_hbm, t_hbm, out_hbm, tv, plane_v, out_v):
    cid = lax.axis_index("c")
    sid = lax.axis_index("s")
    wid = sid * 2 + cid
    b = wid // WORKERS_PER_IMAGE
    sub = wid % WORKERS_PER_IMAGE

    pltpu.sync_copy(t_hbm.at[b], tv)
    tvec = tv[...]

    i00 = _bf16_round(_splat(tvec, 0))
    i01 = _bf16_round(_splat(tvec, 1))
    i10 = _bf16_round(_splat(tvec, 2))
    i11 = _bf16_round(_splat(tvec, 3))
    ntx = _bf16_round(-_splat(tvec, 4))
    nty = _bf16_round(-_splat(tvec, 5))
    tpx = ntx * i00 + nty * i10
    tpy = ntx * i01 + nty * i11

    lim = jnp.float32(H - 2)
    zero = jnp.float32(0.0)
    cx0 = jnp.clip(tpx, zero, lim)
    cy0 = jnp.clip(tpy, zero, lim)
    dx0 = cx0 - cx0.astype(jnp.int32).astype(jnp.float32)
    dy0 = cy0 - cy0.astype(jnp.int32).astype(jnp.float32)
    w00 = (1.0 - dx0) * (1.0 - dy0)
    w10 = dx0 * (1.0 - dy0)
    w01 = (1.0 - dx0) * dy0
    w11 = dx0 * dy0

    lanes_f = lax.iota(jnp.int32, NLANES).astype(jnp.float32)

    def chan_body(k, carry):
        plane = b * C + sub * CH_PER_WORKER + k
        pltpu.sync_copy(x_hbm.at[plane], plane_v)

        def row_body(y, carry2):
            yf = jnp.full((NLANES,), y, dtype=jnp.int32).astype(jnp.float32)
            for j in range(CHUNKS_PER_ROW):
                xf = lanes_f + jnp.float32(j * NLANES)
                cx = jnp.clip((xf * i00 + yf * i10) + tpx, zero, lim)
                cy = jnp.clip((xf * i01 + yf * i11) + tpy, zero, lim)
                ix = cx.astype(jnp.int32)
                iy = cy.astype(jnp.int32)
                f = iy * W + ix
                g00 = plsc.load_gather(plane_v, [f])
                g10 = plsc.load_gather(plane_v, [f + 1])
                g01 = plsc.load_gather(plane_v, [f + W])
                g11 = plsc.load_gather(plane_v, [f + (W + 1)])
                val = w00 * g00 + w10 * g10 + w01 * g01 + w11 * g11
                out_v[pl.ds(y * W + j * NLANES, NLANES)] = val
            return carry2

        lax.fori_loop(0, H, row_body, 0)
        pltpu.sync_copy(out_v, out_hbm.at[plane])
        return carry

    lax.fori_loop(0, CH_PER_WORKER, chan_body, 0)


_affine_sc = functools.partial(
    pl.kernel,
    mesh=plsc.VectorSubcoreMesh(core_axis_name="c", subcore_axis_name="s"),
    out_type=jax.ShapeDtypeStruct((NPLANES, NPIX), jnp.float32),
    compiler_params=pltpu.CompilerParams(needs_layout_passes=False),
    scratch_types=[
        pltpu.VMEM((NLANES,), jnp.float32),
        pltpu.VMEM((NPIX,), jnp.float32),
        pltpu.VMEM((NPIX,), jnp.float32),
    ],
)(_affine_body)


@jax.jit
def kernel(x, transform):
    x2 = x.reshape(NPLANES, NPIX)
    ainv = jnp.linalg.inv(transform[:, :4].reshape(B, 2, 2))
    params = jnp.concatenate([ainv.reshape(B, 4), transform[:, 4:6]], axis=1)
    params = jnp.pad(params, ((0, 0), (0, NLANES - 6)))
    out = _affine_sc(x2, params)
    return out.reshape(x.shape)
